# baseline (speedup 1.0000x reference)
import dataclasses

import jax
import jax.numpy as jnp
from jax import lax
from jax.experimental import pallas as pl
from jax.experimental.pallas import tpu as pltpu
from jax.experimental.pallas import tpu_sc as plsc

_USER_NUM = 1000000
_B = 16384
_F = 16
_NC = 2
_NS = 16
_NW = _NC * _NS
_BPW = _B // _NW
_CHUNK = 128
_NCH = _BPW // _CHUNK
_LANES = 16


def _fm_sc_body(user_ref, item_ref, emb_ref, lin_ref, bias_ref, out_ref,
                uidx, iidx, urows, irows, ulin, ilin, outv, biasv, sem):
    wid = lax.axis_index("s") * _NC + lax.axis_index("c")
    row0 = wid * _NCH
    base = wid * _BPW

    pltpu.sync_copy(user_ref.at[pl.ds(row0, _NCH)], uidx)
    pltpu.sync_copy(item_ref.at[pl.ds(row0, _NCH)], iidx)
    pltpu.sync_copy(bias_ref, biasv)

    for j in range(_NCH):
        for c in range(_CHUNK // _LANES):
            sl = (j, pl.ds(c * _LANES, _LANES))
            iidx[sl] = iidx[sl] + _USER_NUM

    copies = []
    for j in range(_NCH):
        rsl = pl.ds(j * _CHUNK, _CHUNK)
        copies.append(pltpu.async_copy(emb_ref.at[uidx.at[j]], urows.at[rsl], sem))
        copies.append(pltpu.async_copy(emb_ref.at[iidx.at[j]], irows.at[rsl], sem))
        copies.append(pltpu.async_copy(lin_ref.at[uidx.at[j]], ulin.at[rsl], sem))
        copies.append(pltpu.async_copy(lin_ref.at[iidx.at[j]], ilin.at[rsl], sem))
    for cp in copies:
        cp.wait()

    b = biasv[...]

    @pl.loop(0, _BPW, step=_LANES)
    def _(r0):
        ridx = r0 + jnp.arange(_LANES, dtype=jnp.int32)
        acc = jnp.zeros((_LANES,), jnp.float32)
        for f in range(_F):
            fidx = jnp.full((_LANES,), f, jnp.int32)
            a = plsc.load_gather(urows, [ridx, fidx])
            c = plsc.load_gather(irows, [ridx, fidx])
            acc = acc + a * c
        osl = pl.ds(r0, _LANES)
        outv[osl] = acc + ulin[osl] + ilin[osl] + b

    pltpu.sync_copy(outv, out_ref.at[pl.ds(base, _BPW)])


def kernel(user, item, emb_table, lin_table, bias):
    user2 = user.reshape(_NW * _NCH, _CHUNK)
    item2 = item.reshape(_NW * _NCH, _CHUNK)
    lin2 = lin_table.reshape(-1)
    bias16 = jnp.broadcast_to(bias, (_LANES,))
    mesh = plsc.VectorSubcoreMesh(core_axis_name="c", subcore_axis_name="s")
    cp = pltpu.CompilerParams()
    for fld, val in (("needs_layout_passes", False),
                     ("use_tc_tiling_on_sc", False)):
        if fld in pltpu.CompilerParams.__dataclass_fields__:
            cp = dataclasses.replace(cp, **{fld: val})
    f = pl.kernel(
        _fm_sc_body,
        out_type=jax.ShapeDtypeStruct((_B,), jnp.float32),
        mesh=mesh,
        scratch_types=[
            pltpu.VMEM((_NCH, _CHUNK), jnp.int32),
            pltpu.VMEM((_NCH, _CHUNK), jnp.int32),
            pltpu.VMEM((_BPW, _F), jnp.float32),
            pltpu.VMEM((_BPW, _F), jnp.float32),
            pltpu.VMEM((_BPW,), jnp.float32),
            pltpu.VMEM((_BPW,), jnp.float32),
            pltpu.VMEM((_BPW,), jnp.float32),
            pltpu.VMEM((_LANES,), jnp.float32),
            pltpu.SemaphoreType.DMA,
        ],
        compiler_params=cp,
    )
    return f(user2, item2, emb_table, lin2, bias16)
